# Initial kernel scaffold; baseline (speedup 1.0000x reference)
import jax
import jax.numpy as jnp
from jax.experimental import pallas as pl


def kernel(x):
    raise NotImplementedError("write your pallas kernel here")


# speedup vs baseline: 18.6350x; 18.6350x over previous
import functools

import jax
import jax.numpy as jnp
from jax.experimental import pallas as pl

_K = 256


def _topk_mask_kernel(x_ref, o_ref):
    x = x_ref[...]
    u = jax.lax.bitcast_convert_type(x, jnp.uint32)
    sign = u >> 31
    flip = (sign * jnp.uint32(0x7FFFFFFF)) | jnp.uint32(0x80000000)
    key = u ^ flip

    rows = x.shape[0]
    t = jnp.zeros((rows, 1), dtype=jnp.uint32)
    for b in range(31, -1, -1):
        cand = t | jnp.uint32(1 << b)
        cnt = jnp.sum((key >= cand).astype(jnp.int32), axis=1, keepdims=True)
        t = jnp.where(cnt >= _K, cand, t)

    o_ref[...] = (key >= t).astype(jnp.float32)


@jax.jit
def kernel(x):
    n_rows, n_cols = x.shape
    block_rows = 8
    return pl.pallas_call(
        _topk_mask_kernel,
        grid=(n_rows // block_rows,),
        in_specs=[pl.BlockSpec((block_rows, n_cols), lambda i: (i, 0))],
        out_specs=pl.BlockSpec((block_rows, n_cols), lambda i: (i, 0)),
        out_shape=jax.ShapeDtypeStruct((n_rows, n_cols), jnp.float32),
    )(x)
